# baseline (speedup 1.0000x reference)
import functools

import jax
import jax.numpy as jnp
from jax import lax
from jax.experimental import pallas as pl
from jax.experimental.pallas import tpu as pltpu
from jax.experimental.pallas import tpu_sc as plsc

N_BINS = 255
N_ROWS = 1000000
N_FEATURES = 32
TOTAL = N_ROWS * N_FEATURES
NUM_WORKERS = 32
PER_WORKER = TOTAL // NUM_WORKERS
CHUNK = 20000
NUM_CHUNKS = PER_WORKER // CHUNK
PAIRS = CHUNK // 32


def _body(x_hbm, min_hbm, scale_hbm, out_hbm, min_v, scale_v, in_v, out_v):
    wid = lax.axis_index("s") * 2 + lax.axis_index("c")
    pltpu.sync_copy(min_hbm, min_v)
    pltpu.sync_copy(scale_hbm, scale_v)
    m0 = min_v[pl.ds(0, 16)]
    m1 = min_v[pl.ds(16, 16)]
    s0 = scale_v[pl.ds(0, 16)]
    s1 = scale_v[pl.ds(16, 16)]

    def chunk_body(c, carry):
        base = wid * PER_WORKER + c * CHUNK
        pltpu.sync_copy(x_hbm.at[pl.ds(base, CHUNK)], in_v)

        def pair_body(j, carry2):
            off = j * 32
            x0 = in_v[pl.ds(off, 16)]
            x1 = in_v[pl.ds(off + 16, 16)]
            y0 = ((x0 - m0) * s0).astype(jnp.int32)
            y1 = ((x1 - m1) * s1).astype(jnp.int32)
            y0 = jnp.minimum(jnp.maximum(y0, 0), N_BINS - 1)
            y1 = jnp.minimum(jnp.maximum(y1, 0), N_BINS - 1)
            out_v[pl.ds(off, 16)] = y0
            out_v[pl.ds(off + 16, 16)] = y1
            return carry2

        lax.fori_loop(0, PAIRS, pair_body, 0)
        pltpu.sync_copy(out_v, out_hbm.at[pl.ds(base, CHUNK)])
        return carry

    lax.fori_loop(0, NUM_CHUNKS, chunk_body, 0)


@jax.jit
def _discretize(x_flat, tmin, scale):
    mesh = plsc.VectorSubcoreMesh(core_axis_name="c", subcore_axis_name="s")
    f = pl.kernel(
        _body,
        out_type=jax.ShapeDtypeStruct((TOTAL,), jnp.int32),
        mesh=mesh,
        scratch_types=[
            pltpu.VMEM((N_FEATURES,), jnp.float32),
            pltpu.VMEM((N_FEATURES,), jnp.float32),
            pltpu.VMEM((CHUNK,), jnp.float32),
            pltpu.VMEM((CHUNK,), jnp.int32),
        ],
    )
    return f(x_flat, tmin, scale)


def kernel(X, tensor_min, tensor_max):
    scale = N_BINS / (tensor_max - tensor_min)
    out = _discretize(X.reshape(-1), tensor_min, scale)
    return out.reshape(N_ROWS, N_FEATURES)


# speedup vs baseline: 1.1464x; 1.1464x over previous
import jax
import jax.numpy as jnp
from jax import lax
from jax.experimental import pallas as pl
from jax.experimental.pallas import tpu as pltpu
from jax.experimental.pallas import tpu_sc as plsc

N_BINS = 255
N_ROWS = 1000000
N_FEATURES = 32
TOTAL = N_ROWS * N_FEATURES
NUM_WORKERS = 32
PER_WORKER = TOTAL // NUM_WORKERS
CHUNK = 20000
NUM_CHUNKS = PER_WORKER // CHUNK
PAIRS = CHUNK // 32
NBUF = 2


def _body(x_hbm, min_hbm, scale_hbm, out_hbm,
          min_v, scale_v, in0, in1, out0, out1,
          in_sem0, in_sem1, out_sem0, out_sem1):
    wid = lax.axis_index("s") * 2 + lax.axis_index("c")
    base0 = wid * PER_WORKER
    pltpu.sync_copy(min_hbm, min_v)
    pltpu.sync_copy(scale_hbm, scale_v)
    m0 = min_v[pl.ds(0, 16)]
    m1 = min_v[pl.ds(16, 16)]
    s0 = scale_v[pl.ds(0, 16)]
    s1 = scale_v[pl.ds(16, 16)]

    in_bufs = (in0, in1)
    out_bufs = (out0, out1)
    in_sems = (in_sem0, in_sem1)
    out_sems = (out_sem0, out_sem1)

    for b in range(NBUF):
        pltpu.async_copy(
            x_hbm.at[pl.ds(base0 + b * CHUNK, CHUNK)], in_bufs[b], in_sems[b])

    def outer(g, carry):
        for b in range(NBUF):
            c = g * NBUF + b
            in_b, out_b = in_bufs[b], out_bufs[b]
            cbase = base0 + c * CHUNK
            pltpu.make_async_copy(
                x_hbm.at[pl.ds(0, CHUNK)], in_b, in_sems[b]).wait()
            @pl.when(c >= NBUF)
            def _():
                pltpu.make_async_copy(
                    out_b, out_hbm.at[pl.ds(0, CHUNK)], out_sems[b]).wait()

            @plsc.parallel_loop(0, PAIRS, unroll=5)
            def _(j):
                off = j * 32
                x0 = in_b[pl.ds(off, 16)]
                x1 = in_b[pl.ds(off + 16, 16)]
                y0 = ((x0 - m0) * s0).astype(jnp.int32)
                y1 = ((x1 - m1) * s1).astype(jnp.int32)
                y0 = jnp.minimum(jnp.maximum(y0, 0), N_BINS - 1)
                y1 = jnp.minimum(jnp.maximum(y1, 0), N_BINS - 1)
                out_b[pl.ds(off, 16)] = y0
                out_b[pl.ds(off + 16, 16)] = y1

            pltpu.async_copy(
                out_b, out_hbm.at[pl.ds(cbase, CHUNK)], out_sems[b])
            @pl.when(c + NBUF < NUM_CHUNKS)
            def _():
                pltpu.async_copy(
                    x_hbm.at[pl.ds(cbase + NBUF * CHUNK, CHUNK)],
                    in_b, in_sems[b])
        return carry

    lax.fori_loop(0, NUM_CHUNKS // NBUF, outer, 0)

    for b in range(NBUF):
        pltpu.make_async_copy(
            out_bufs[b], out_hbm.at[pl.ds(0, CHUNK)], out_sems[b]).wait()


@jax.jit
def _discretize(x_flat, tmin, scale):
    mesh = plsc.VectorSubcoreMesh(core_axis_name="c", subcore_axis_name="s")
    f = pl.kernel(
        _body,
        out_type=jax.ShapeDtypeStruct((TOTAL,), jnp.int32),
        mesh=mesh,
        scratch_types=[
            pltpu.VMEM((N_FEATURES,), jnp.float32),
            pltpu.VMEM((N_FEATURES,), jnp.float32),
            pltpu.VMEM((CHUNK,), jnp.float32),
            pltpu.VMEM((CHUNK,), jnp.float32),
            pltpu.VMEM((CHUNK,), jnp.int32),
            pltpu.VMEM((CHUNK,), jnp.int32),
            pltpu.SemaphoreType.DMA,
            pltpu.SemaphoreType.DMA,
            pltpu.SemaphoreType.DMA,
            pltpu.SemaphoreType.DMA,
        ],
    )
    return f(x_flat, tmin, scale)


def kernel(X, tensor_min, tensor_max):
    scale = N_BINS / (tensor_max - tensor_min)
    out = _discretize(X.reshape(-1), tensor_min, scale)
    return out.reshape(N_ROWS, N_FEATURES)


# speedup vs baseline: 1.7964x; 1.5669x over previous
import jax
import jax.numpy as jnp
from jax import lax
from jax.experimental import pallas as pl
from jax.experimental.pallas import tpu as pltpu
from jax.experimental.pallas import tpu_sc as plsc

N_BINS = 255
N_ROWS = 1000000
N_FEATURES = 32
NUM_WORKERS = 32
CHUNK_ROWS = 200
NUM_CHUNKS = N_ROWS // CHUNK_ROWS
MAX_PER_W = -(-NUM_CHUNKS // NUM_WORKERS)
NBUF = 2


def _body(x_hbm, min_hbm, scale_hbm, out_hbm,
          min_v, scale_v, in0, in1, out0, out1,
          in_sem0, in_sem1, out_sem0, out_sem1):
    wid = lax.axis_index("s") * 2 + lax.axis_index("c")
    pltpu.sync_copy(min_hbm, min_v)
    pltpu.sync_copy(scale_hbm, scale_v)
    m0 = min_v[pl.ds(0, 16)]
    m1 = min_v[pl.ds(16, 16)]
    s0 = scale_v[pl.ds(0, 16)]
    s1 = scale_v[pl.ds(16, 16)]

    in_bufs = (in0, in1)
    out_bufs = (out0, out1)
    in_sems = (in_sem0, in_sem1)
    out_sems = (out_sem0, out_sem1)

    n_w = (NUM_CHUNKS - wid + NUM_WORKERS - 1) // NUM_WORKERS

    for b in range(NBUF):
        @pl.when(b < n_w)
        def _():
            pltpu.async_copy(
                x_hbm.at[wid + b * NUM_WORKERS], in_bufs[b], in_sems[b])

    def outer(g, carry):
        for b in range(NBUF):
            i = g * NBUF + b
            in_b, out_b = in_bufs[b], out_bufs[b]

            @pl.when(i < n_w)
            def _():
                c = wid + i * NUM_WORKERS
                pltpu.make_async_copy(
                    x_hbm.at[0], in_b, in_sems[b]).wait()

                @pl.when(i >= NBUF)
                def _():
                    pltpu.make_async_copy(
                        out_b, out_hbm.at[0], out_sems[b]).wait()

                @plsc.parallel_loop(0, CHUNK_ROWS, unroll=8)
                def _(j):
                    x0 = in_b[j, pl.ds(0, 16)]
                    x1 = in_b[j, pl.ds(16, 16)]
                    y0 = ((x0 - m0) * s0).astype(jnp.int32)
                    y1 = ((x1 - m1) * s1).astype(jnp.int32)
                    y0 = jnp.minimum(jnp.maximum(y0, 0), N_BINS - 1)
                    y1 = jnp.minimum(jnp.maximum(y1, 0), N_BINS - 1)
                    out_b[j, pl.ds(0, 16)] = y0
                    out_b[j, pl.ds(16, 16)] = y1

                pltpu.async_copy(out_b, out_hbm.at[c], out_sems[b])

                @pl.when(i + NBUF < n_w)
                def _():
                    pltpu.async_copy(
                        x_hbm.at[c + NBUF * NUM_WORKERS], in_b, in_sems[b])
        return carry

    lax.fori_loop(0, (MAX_PER_W + NBUF - 1) // NBUF, outer, 0)

    for b in range(NBUF):
        @pl.when(jnp.logical_and(n_w >= 1, (n_w - 1) % NBUF == b)
                 | jnp.logical_and(n_w >= 2, (n_w - 2) % NBUF == b))
        def _():
            pltpu.make_async_copy(
                out_bufs[b], out_hbm.at[0], out_sems[b]).wait()


@jax.jit
def _discretize(x, tmin, scale):
    mesh = plsc.VectorSubcoreMesh(core_axis_name="c", subcore_axis_name="s")
    f = pl.kernel(
        _body,
        out_type=jax.ShapeDtypeStruct(
            (NUM_CHUNKS, CHUNK_ROWS, N_FEATURES), jnp.int32),
        mesh=mesh,
        scratch_types=[
            pltpu.VMEM((N_FEATURES,), jnp.float32),
            pltpu.VMEM((N_FEATURES,), jnp.float32),
            pltpu.VMEM((CHUNK_ROWS, N_FEATURES), jnp.float32),
            pltpu.VMEM((CHUNK_ROWS, N_FEATURES), jnp.float32),
            pltpu.VMEM((CHUNK_ROWS, N_FEATURES), jnp.int32),
            pltpu.VMEM((CHUNK_ROWS, N_FEATURES), jnp.int32),
            pltpu.SemaphoreType.DMA,
            pltpu.SemaphoreType.DMA,
            pltpu.SemaphoreType.DMA,
            pltpu.SemaphoreType.DMA,
        ],
    )
    return f(x.reshape(NUM_CHUNKS, CHUNK_ROWS, N_FEATURES), tmin, scale)


def kernel(X, tensor_min, tensor_max):
    scale = N_BINS / (tensor_max - tensor_min)
    out = _discretize(X, tensor_min, scale)
    return out.reshape(N_ROWS, N_FEATURES)
